# baseline (speedup 1.0000x reference)
import functools

import jax
import jax.numpy as jnp
from jax import lax
from jax.experimental import pallas as pl
from jax.experimental.pallas import tpu as pltpu
from jax.experimental.pallas import tpu_sc as plsc

_LANES = 16
_NSETS = 3


def kernel(input_ids, word_embeddings, position_embeddings):
    batch, seqlen = input_ids.shape
    _, dim = word_embeddings.shape

    num_cores, num_subcores = 2, 16
    num_workers = num_cores * num_subcores
    pos_per_worker = seqlen // num_workers
    chunk = 8
    num_chunks = pos_per_worker // chunk
    rows = batch * chunk

    mesh = plsc.VectorSubcoreMesh(core_axis_name="c", subcore_axis_name="s")

    @functools.partial(
        pl.kernel,
        out_type=jax.ShapeDtypeStruct((batch, seqlen, dim), jnp.float32),
        mesh=mesh,
        scratch_types=(
            [pltpu.VMEM((num_chunks * rows,), jnp.int32)]
            + [pltpu.VMEM((rows, dim), jnp.float32)] * _NSETS
            + [pltpu.VMEM((chunk, dim), jnp.float32)] * _NSETS
            + [pltpu.SemaphoreType.DMA] * _NSETS
            + [pltpu.SemaphoreType.DMA] * _NSETS
            + [pltpu.SemaphoreType.DMA] * _NSETS
            + [pltpu.SemaphoreType.DMA]
        ),
    )
    def emb_kernel(ids_hbm, word_hbm, pos_hbm, out_hbm, idx_v, *rest):
        wbuf = rest[0:_NSETS]
        pbuf = rest[_NSETS:2 * _NSETS]
        gsem = rest[2 * _NSETS:3 * _NSETS]
        osem = rest[3 * _NSETS:4 * _NSETS]
        psem = rest[4 * _NSETS:5 * _NSETS]
        isem = rest[5 * _NSETS]

        wid = lax.axis_index("s") * num_cores + lax.axis_index("c")
        pos_base = wid * pos_per_worker

        def pos_desc(ci, s):
            src = pos_hbm.at[pl.ds(pos_base + ci * chunk, chunk)]
            return pltpu.make_async_copy(src, pbuf[s], psem[s])

        def gather_desc(ci, s):
            src = word_hbm.at[idx_v.at[pl.ds(ci * rows, rows)]]
            return pltpu.make_async_copy(src, wbuf[s], gsem[s])

        def out_drain(s):
            dst = out_hbm.at[0, pl.ds(0, rows)]
            pltpu.make_async_copy(wbuf[s], dst, osem[s]).wait()

        for ci in range(num_chunks):
            for b in range(batch):
                pltpu.make_async_copy(
                    ids_hbm.at[b, pl.ds(pos_base + ci * chunk, chunk)],
                    idx_v.at[pl.ds(ci * rows + b * chunk, chunk)],
                    isem).start()
        pltpu.make_async_copy(
            ids_hbm.at[0, pl.ds(0, num_chunks * rows)], idx_v, isem).wait()

        pos_desc(0, 0).start()
        gather_desc(0, 0).start()

        def step(ci, s):
            snext = (s + 1) % _NSETS

            @pl.when(ci + 1 < num_chunks)
            def _refill():
                @pl.when(ci >= _NSETS - 1)
                def _drain():
                    out_drain(snext)
                gather_desc(ci + 1, snext).start()
                pos_desc(ci + 1, snext).start()

            gather_desc(ci, s).wait()
            pos_desc(ci, s).wait()

            for b in range(batch):
                @pl.loop(0, chunk)
                def _row(r):
                    for j in range(dim // _LANES):
                        sl = pl.ds(j * _LANES, _LANES)
                        plsc.addupdate(wbuf[s].at[b * chunk + r, sl],
                                       pbuf[s][r, sl])

                dst = out_hbm.at[b, pl.ds(pos_base + ci * chunk, chunk)]
                pltpu.make_async_copy(
                    wbuf[s].at[pl.ds(b * chunk, chunk)], dst, osem[s]).start()

        main = (num_chunks // _NSETS) * _NSETS

        @pl.loop(0, main, step=_NSETS)
        def _chunks(cio):
            for si in range(_NSETS):
                step(cio + si, si)

        for ci in range(main, num_chunks):
            step(ci, ci % _NSETS)

        for ci in range(num_chunks - _NSETS, num_chunks):
            out_drain(ci % _NSETS)

    return emb_kernel(input_ids, word_embeddings, position_embeddings)


# speedup vs baseline: 1.0529x; 1.0529x over previous
import functools

import jax
import jax.numpy as jnp
from jax import lax
from jax.experimental import pallas as pl
from jax.experimental.pallas import tpu as pltpu
from jax.experimental.pallas import tpu_sc as plsc

_LANES = 16
_NSETS = 3


def kernel(input_ids, word_embeddings, position_embeddings):
    batch, seqlen = input_ids.shape
    _, dim = word_embeddings.shape

    num_cores, num_subcores = 2, 16
    num_workers = num_cores * num_subcores
    pos_per_worker = seqlen // num_workers
    chunk = 8
    num_chunks = pos_per_worker // chunk
    rows = batch * chunk

    mesh = plsc.VectorSubcoreMesh(core_axis_name="c", subcore_axis_name="s")

    @functools.partial(
        pl.kernel,
        out_type=jax.ShapeDtypeStruct((batch, seqlen, dim), jnp.float32),
        mesh=mesh,
        scratch_types=(
            [pltpu.VMEM((num_chunks * rows,), jnp.int32)]
            + [pltpu.VMEM((rows, dim), jnp.float32)] * _NSETS
            + [pltpu.VMEM((chunk, dim), jnp.float32)] * _NSETS
            + [pltpu.SemaphoreType.DMA] * _NSETS
            + [pltpu.SemaphoreType.DMA] * _NSETS
            + [pltpu.SemaphoreType.DMA] * _NSETS
            + [pltpu.SemaphoreType.DMA]
        ),
    )
    def emb_kernel(ids_hbm, word_hbm, pos_hbm, out_hbm, idx_v, *rest):
        wbuf = rest[0:_NSETS]
        pbuf = rest[_NSETS:2 * _NSETS]
        gsem = rest[2 * _NSETS:3 * _NSETS]
        osem = rest[3 * _NSETS:4 * _NSETS]
        psem = rest[4 * _NSETS:5 * _NSETS]
        isem = rest[5 * _NSETS]

        wid = lax.axis_index("s") * num_cores + lax.axis_index("c")
        pos_base = wid * pos_per_worker

        def pos_desc(ci, s):
            src = pos_hbm.at[pl.ds(pos_base + ci * chunk, chunk)]
            return pltpu.make_async_copy(src, pbuf[s], psem[s])

        def gather_desc(ci, s):
            src = word_hbm.at[idx_v.at[pl.ds(ci * rows, rows)]]
            return pltpu.make_async_copy(src, wbuf[s], gsem[s])

        def out_start(ci, s):
            for b in range(batch):
                dst = out_hbm.at[b, pl.ds(pos_base + ci * chunk, chunk)]
                pltpu.make_async_copy(
                    wbuf[s].at[pl.ds(b * chunk, chunk)], dst, osem[s]).start()

        def out_drain(s):
            dst = out_hbm.at[0, pl.ds(0, rows)]
            pltpu.make_async_copy(wbuf[s], dst, osem[s]).wait()

        for ci in range(num_chunks):
            for b in range(batch):
                pltpu.make_async_copy(
                    ids_hbm.at[b, pl.ds(pos_base + ci * chunk, chunk)],
                    idx_v.at[pl.ds(ci * rows + b * chunk, chunk)],
                    isem).start()
        pltpu.make_async_copy(
            ids_hbm.at[0, pl.ds(0, num_chunks * rows)], idx_v, isem).wait()

        pos_desc(0, 0).start()
        gather_desc(0, 0).start()

        def step(ci, s):
            snext = (s + 1) % _NSETS

            @pl.when(ci + 1 < num_chunks)
            def _refill():
                @pl.when(ci >= _NSETS - 1)
                def _drain():
                    out_drain(snext)
                gather_desc(ci + 1, snext).start()
                pos_desc(ci + 1, snext).start()

            gather_desc(ci, s).wait()
            pos_desc(ci, s).wait()

            @pl.loop(0, chunk)
            def _row(r):
                for j in range(dim // _LANES):
                    sl = pl.ds(j * _LANES, _LANES)
                    x = pbuf[s][r, sl]
                    for b in range(batch):
                        plsc.addupdate(wbuf[s].at[b * chunk + r, sl], x)

            out_start(ci, s)

        main = (num_chunks // _NSETS) * _NSETS

        @pl.loop(0, main, step=_NSETS)
        def _chunks(cio):
            for si in range(_NSETS):
                step(cio + si, si)

        for ci in range(main, num_chunks):
            step(ci, ci % _NSETS)

        for ci in range(num_chunks - _NSETS, num_chunks):
            out_drain(ci % _NSETS)

    return emb_kernel(input_ids, word_embeddings, position_embeddings)
